# baseline (speedup 1.0000x reference)
import functools

import jax
import jax.numpy as jnp
from jax import lax
from jax.experimental import pallas as pl
from jax.experimental.pallas import tpu as pltpu
from jax.experimental.pallas import tpu_sc as plsc

B = 4096
N_DENSE = 13
N_SPARSE = 26
VOCAB = 1000
EDIM = 128
EPS = 1e-3
ROWS = B * N_SPARSE
NW = 32
RPW = ROWS // NW
CH = 128
NCHUNK = RPW // CH
NBUF = 2
NSPLIT = 2


@functools.cache
def _make_sc_gather(nb):
    ncw = (N_SPARSE * nb // CH) // NW
    ncw_pad = -(-ncw // 8) * 8
    mesh = plsc.VectorSubcoreMesh(core_axis_name="c", subcore_axis_name="s")
    return pl.kernel(
        functools.partial(_sc_gather_body, nb=nb, ncw=ncw),
        out_type=jax.ShapeDtypeStruct((nb, N_SPARSE * EDIM), jnp.float32),
        mesh=mesh,
        scratch_types=[
            pltpu.VMEM((ncw_pad, CH), jnp.int32),
            pltpu.VMEM((CH, EDIM), jnp.float32),
            pltpu.VMEM((CH, EDIM), jnp.float32),
            pltpu.SemaphoreType.DMA,
            pltpu.SemaphoreType.DMA,
        ],
    )


def _sc_gather_body(table_hbm, idx_hbm, out_hbm, idx_v, buf0, buf1, sem0,
                    sem1, *, nb, ncw):
    wid = lax.axis_index("s") * 2 + lax.axis_index("c")
    pltpu.sync_copy(idx_hbm.at[wid], idx_v)
    bufs = (buf0, buf1)
    sems = (sem0, sem1)

    def chunk_out(j):
        c = wid * ncw + j
        field = c // (nb // CH)
        brow = (c % (nb // CH)) * CH
        return out_hbm.at[pl.ds(brow, CH), pl.ds(field * EDIM, EDIM)]

    for b in range(NBUF):
        pltpu.async_copy(table_hbm.at[idx_v.at[b]], bufs[b], sems[b])

    n_main = (ncw // NBUF) * NBUF

    def outer(j0, carry):
        for b in range(NBUF):
            j = j0 * NBUF + b
            pltpu.make_async_copy(table_hbm.at[idx_v.at[b]], bufs[b],
                                  sems[b]).wait()
            pltpu.sync_copy(bufs[b], chunk_out(j))

            @pl.when(j + NBUF < ncw)
            def _():
                pltpu.async_copy(table_hbm.at[idx_v.at[j + NBUF]], bufs[b],
                                 sems[b])
        return carry

    lax.fori_loop(0, n_main // NBUF, outer, 0)
    for j in range(n_main, ncw):
        b = j % NBUF
        pltpu.make_async_copy(table_hbm.at[idx_v.at[b]], bufs[b],
                              sems[b]).wait()
        pltpu.sync_copy(bufs[b], chunk_out(j))


_S = float(1.0 / (1.0 + EPS) ** 0.5)


def _mlp_body(inp_ref, emb_ref, w_ref, W1s_ref, W1d_ref, P1_ref, W2_ref,
              P2_ref, W3_ref, P3_ref, Wo_ref, wb_ref, out_ref):
    f32 = jnp.float32
    bf16 = jnp.bfloat16
    inp = inp_ref[...]
    h = jnp.dot(emb_ref[...].astype(bf16), W1s_ref[...],
                preferred_element_type=f32)
    h = h + jnp.dot(inp, W1d_ref[...], preferred_element_type=f32)
    h = jnp.maximum((h + P1_ref[0:1, :]) * (P1_ref[1:2, :] * _S)
                    + P1_ref[2:3, :], 0.0)
    h = jnp.dot(h.astype(bf16), W2_ref[...], preferred_element_type=f32)
    h = jnp.maximum((h + P2_ref[0:1, :]) * (P2_ref[1:2, :] * _S)
                    + P2_ref[2:3, :], 0.0)
    h = jnp.dot(h.astype(bf16), W3_ref[...], preferred_element_type=f32)
    h = jnp.maximum((h + P3_ref[0:1, :]) * (P3_ref[1:2, :] * _S)
                    + P3_ref[2:3, :], 0.0)
    deep = jnp.dot(h.astype(bf16), Wo_ref[...], preferred_element_type=f32)
    wide = jnp.dot(inp, w_ref[...], preferred_element_type=f32)
    out_ref[...] = jax.nn.sigmoid(deep + wide + wb_ref[...])


def _mlp_call(inputs_pad, emb, w_ext, W1s, W1d_ext, P1, W2, P2, W3, P3,
              Wo, wb):
    nb = inputs_pad.shape[0]
    blk = 512
    grid = (nb // blk,)
    full = lambda a: pl.BlockSpec(a.shape, lambda i: (0,) * a.ndim)
    in_specs = [
        pl.BlockSpec((blk, 128), lambda i: (i, 0)),
        pl.BlockSpec((blk, N_SPARSE * EDIM), lambda i: (i, 0)),
        full(w_ext), full(W1s), full(W1d_ext), full(P1), full(W2),
        full(P2), full(W3), full(P3), full(Wo), full(wb),
    ]
    return pl.pallas_call(
        _mlp_body,
        grid=grid,
        in_specs=in_specs,
        out_specs=pl.BlockSpec((blk, 1), lambda i: (i, 0)),
        out_shape=jax.ShapeDtypeStruct((nb, 1), jnp.float32),
    )(inputs_pad, emb, w_ext, W1s, W1d_ext, P1, W2, P2, W3, P3, Wo, wb)


def kernel(inputs, tables, w, b, W1, B1, g1, be1, W2, B2, g2, be2, W3, B3,
           g3, be3, Wo, Bo):
    idx = inputs[:, N_DENSE:].astype(jnp.int32)
    flat_idx = (idx + jnp.arange(N_SPARSE, dtype=jnp.int32)[None, :] * VOCAB)
    tables_flat = tables.reshape(N_SPARSE * VOCAB, EDIM)

    inputs_pad = jnp.pad(inputs, ((0, 0), (0, 128 - (N_DENSE + N_SPARSE))))
    w_ext = jnp.pad(w, ((0, 128 - (N_DENSE + N_SPARSE)), (0, 0)))
    W1s = W1[: N_SPARSE * EDIM].astype(jnp.bfloat16)
    W1d_ext = jnp.pad(W1[N_SPARSE * EDIM:], ((0, 128 - N_DENSE), (0, 0)))
    P1 = jnp.stack([B1, g1, be1])
    P2 = jnp.stack([B2, g2, be2])
    P3 = jnp.stack([B3, g3, be3])
    wb = (b + Bo).reshape(1, 1)
    W2b = W2.astype(jnp.bfloat16)
    W3b = W3.astype(jnp.bfloat16)
    Wob = Wo.astype(jnp.bfloat16)

    nb = B // NSPLIT
    ncw = (N_SPARSE * nb // CH) // NW
    ncw_pad = -(-ncw // 8) * 8
    gather = _make_sc_gather(nb)
    outs = []
    for h in range(NSPLIT):
        idx_h = flat_idx[h * nb:(h + 1) * nb].T.reshape(NW, ncw, CH)
        idx_h = jnp.pad(idx_h, ((0, 0), (0, ncw_pad - ncw), (0, 0)))
        emb_h = gather(tables_flat, idx_h)
        outs.append(_mlp_call(inputs_pad[h * nb:(h + 1) * nb], emb_h, w_ext,
                              W1s, W1d_ext, P1, W2b, P2, W3b, P3, Wob, wb))
    return jnp.concatenate(outs, axis=0) if NSPLIT > 1 else outs[0]


# speedup vs baseline: 1.0736x; 1.0736x over previous
import functools

import jax
import jax.numpy as jnp
from jax import lax
from jax.experimental import pallas as pl
from jax.experimental.pallas import tpu as pltpu
from jax.experimental.pallas import tpu_sc as plsc

B = 4096
N_DENSE = 13
N_SPARSE = 26
VOCAB = 1000
EDIM = 128
EPS = 1e-3
ROWS = B * N_SPARSE
NW = 32
RPW = ROWS // NW
CH = 128
NCHUNK = RPW // CH
NBUF = 2
NSPLIT = 2


@functools.cache
def _make_sc_gather(nb):
    ncw = (N_SPARSE * nb // CH) // NW
    ncw_pad = -(-ncw // 8) * 8
    mesh = plsc.VectorSubcoreMesh(core_axis_name="c", subcore_axis_name="s")
    return pl.kernel(
        functools.partial(_sc_gather_body, nb=nb, ncw=ncw),
        out_type=jax.ShapeDtypeStruct((nb, N_SPARSE * EDIM), jnp.float32),
        mesh=mesh,
        scratch_types=[
            pltpu.VMEM((ncw_pad, CH), jnp.int32),
            pltpu.VMEM((CH, EDIM), jnp.float32),
            pltpu.VMEM((CH, EDIM), jnp.float32),
            pltpu.SemaphoreType.DMA,
            pltpu.SemaphoreType.DMA,
        ],
    )


def _sc_gather_body(table_hbm, idx_hbm, out_hbm, idx_v, buf0, buf1, sem0,
                    sem1, *, nb, ncw):
    wid = lax.axis_index("s") * 2 + lax.axis_index("c")
    pltpu.sync_copy(idx_hbm.at[wid], idx_v)
    bufs = (buf0, buf1)
    sems = (sem0, sem1)

    def chunk_out(j):
        c = wid * ncw + j
        field = c // (nb // CH)
        brow = (c % (nb // CH)) * CH
        return out_hbm.at[pl.ds(brow, CH), pl.ds(field * EDIM, EDIM)]

    for b in range(NBUF):
        pltpu.async_copy(table_hbm.at[idx_v.at[b]], bufs[b], sems[b])

    n_main = (ncw // NBUF) * NBUF

    def outer(j0, carry):
        for b in range(NBUF):
            j = j0 * NBUF + b
            pltpu.make_async_copy(table_hbm.at[idx_v.at[b]], bufs[b],
                                  sems[b]).wait()
            pltpu.sync_copy(bufs[b], chunk_out(j))

            @pl.when(j + NBUF < ncw)
            def _():
                pltpu.async_copy(table_hbm.at[idx_v.at[j + NBUF]], bufs[b],
                                 sems[b])
        return carry

    lax.fori_loop(0, n_main // NBUF, outer, 0)
    for j in range(n_main, ncw):
        b = j % NBUF
        pltpu.make_async_copy(table_hbm.at[idx_v.at[b]], bufs[b],
                              sems[b]).wait()
        pltpu.sync_copy(bufs[b], chunk_out(j))


_S = float(1.0 / (1.0 + EPS) ** 0.5)


def _mlp_body(inp_ref, emb_ref, w_ref, W1s_ref, W1d_ref, b1_ref, g1_ref,
              e1_ref, W2_ref, b2_ref, g2_ref, e2_ref, W3_ref, b3_ref,
              g3_ref, e3_ref, Wo_ref, wb_ref, out_ref):
    f32 = jnp.float32
    bf16 = jnp.bfloat16
    inp = inp_ref[...]
    h = jnp.dot(emb_ref[...].astype(bf16), W1s_ref[...],
                preferred_element_type=f32)
    h = h + jnp.dot(inp, W1d_ref[...], preferred_element_type=f32)
    h = jnp.maximum((h + b1_ref[...]) * (g1_ref[...] * _S) + e1_ref[...],
                    0.0)
    h = jnp.dot(h.astype(bf16), W2_ref[...], preferred_element_type=f32)
    h = jnp.maximum((h + b2_ref[...]) * (g2_ref[...] * _S) + e2_ref[...],
                    0.0)
    h = jnp.dot(h.astype(bf16), W3_ref[...], preferred_element_type=f32)
    h = jnp.maximum((h + b3_ref[...]) * (g3_ref[...] * _S) + e3_ref[...],
                    0.0)
    deep = jnp.dot(h.astype(bf16), Wo_ref[...], preferred_element_type=f32)
    wide = jnp.dot(inp, w_ref[...], preferred_element_type=f32)
    out_ref[...] = jax.nn.sigmoid(deep + wide + wb_ref[...])


def _mlp_call(inputs, emb, w2d, W1s, W1d_ext, *rest):
    nb = inputs.shape[0]
    blk = 512
    grid = (nb // blk,)
    full = lambda a: pl.BlockSpec(a.shape, lambda i: (0,) * a.ndim)
    in_specs = [
        pl.BlockSpec((blk, N_DENSE + N_SPARSE), lambda i: (i, 0)),
        pl.BlockSpec((blk, N_SPARSE * EDIM), lambda i: (i, 0)),
        full(w2d), full(W1s), full(W1d_ext),
    ] + [full(a) for a in rest]
    return pl.pallas_call(
        _mlp_body,
        grid=grid,
        in_specs=in_specs,
        out_specs=pl.BlockSpec((blk, 1), lambda i: (i, 0)),
        out_shape=jax.ShapeDtypeStruct((nb, 1), jnp.float32),
    )(inputs, emb, w2d, W1s, W1d_ext, *rest)


def kernel(inputs, tables, w, b, W1, B1, g1, be1, W2, B2, g2, be2, W3, B3,
           g3, be3, Wo, Bo):
    idx = inputs[:, N_DENSE:].astype(jnp.int32)
    flat_idx = (idx + jnp.arange(N_SPARSE, dtype=jnp.int32)[None, :] * VOCAB)
    tables_flat = tables.reshape(N_SPARSE * VOCAB, EDIM)

    W1s = W1[: N_SPARSE * EDIM].astype(jnp.bfloat16)
    W1d_ext = jnp.pad(W1[N_SPARSE * EDIM:], ((0, N_SPARSE), (0, 0)))
    vecs = (B1.reshape(1, -1), g1.reshape(1, -1), be1.reshape(1, -1),
            W2.astype(jnp.bfloat16), B2.reshape(1, -1), g2.reshape(1, -1),
            be2.reshape(1, -1), W3.astype(jnp.bfloat16), B3.reshape(1, -1),
            g3.reshape(1, -1), be3.reshape(1, -1), Wo.astype(jnp.bfloat16),
            (b + Bo).reshape(1, 1))

    nb = B // NSPLIT
    ncw = (N_SPARSE * nb // CH) // NW
    ncw_pad = -(-ncw // 8) * 8
    gather = _make_sc_gather(nb)
    outs = []
    for h in range(NSPLIT):
        idx_h = flat_idx[h * nb:(h + 1) * nb].T.reshape(NW, ncw, CH)
        idx_h = jnp.pad(idx_h, ((0, 0), (0, ncw_pad - ncw), (0, 0)))
        emb_h = gather(tables_flat, idx_h)
        outs.append(_mlp_call(inputs[h * nb:(h + 1) * nb], emb_h, w,
                              W1s, W1d_ext, *vecs))
    return jnp.concatenate(outs, axis=0) if NSPLIT > 1 else outs[0]
